# Initial kernel scaffold; baseline (speedup 1.0000x reference)
import jax
import jax.numpy as jnp
from jax.experimental import pallas as pl


def kernel(item_seq, id_table, freq_table, W1, b1, W2, b2, gamma, beta):
    raise NotImplementedError("write your pallas kernel here")


# speedup vs baseline: 2.5523x; 2.5523x over previous
import jax
import jax.numpy as jnp
from jax import lax
from jax.experimental import pallas as pl
from jax.experimental.pallas import tpu as pltpu
from jax.experimental.pallas import tpu_sc as plsc

B, L, H = 4096, 200, 64
N = B * L
NC, NS = 2, 16
NW = NC * NS
PER_W = N // NW
CHUNK = 128
K = PER_W // CHUNK
FW = 8
NUM_ROWS8 = 1000000 * 2 // FW


def _sc_gather_body(idx_hbm, idx4_hbm, id_tab, freq8_tab, id_out, f8_out,
                    idx_v, idx4_v, rows_v, frows_v, sem_id, sem_fr):
    wid = lax.axis_index("s") * NC + lax.axis_index("c")
    pltpu.sync_copy(idx_hbm.at[wid], idx_v)
    pltpu.sync_copy(idx4_hbm.at[wid], idx4_v)

    def body(j, carry):
        base = wid * PER_W + j * CHUNK
        cp1 = pltpu.async_copy(id_tab.at[idx_v.at[j]], rows_v, sem_id)
        cp2 = pltpu.async_copy(freq8_tab.at[idx4_v.at[j]], frows_v, sem_fr)
        cp1.wait()
        pltpu.sync_copy(rows_v, id_out.at[pl.ds(base, CHUNK)])
        cp2.wait()
        pltpu.sync_copy(frows_v, f8_out.at[pl.ds(base, CHUNK)])
        return carry

    lax.fori_loop(0, K, body, 0)


_SC_CACHE = {}


def _sc_gather_call():
    if "k" not in _SC_CACHE:
        _SC_CACHE["k"] = pl.kernel(
            _sc_gather_body,
            out_type=(jax.ShapeDtypeStruct((N, H), jnp.float32),
                      jax.ShapeDtypeStruct((N, FW), jnp.float32)),
            mesh=plsc.VectorSubcoreMesh(core_axis_name="c", subcore_axis_name="s"),
            compiler_params=pltpu.CompilerParams(use_tc_tiling_on_sc=False),
            scratch_types=[
                pltpu.VMEM((K, CHUNK), jnp.int32),
                pltpu.VMEM((K, CHUNK), jnp.int32),
                pltpu.VMEM((CHUNK, H), jnp.float32),
                pltpu.VMEM((CHUNK, FW), jnp.float32),
                pltpu.SemaphoreType.DMA,
                pltpu.SemaphoreType.DMA,
            ],
        )
    return _SC_CACHE["k"]


BT = 2048


def _tc_body(id_ref, f8_ref, m_ref, w1_ref, b1_ref, w2_ref, b2_ref, g_ref,
             be_ref, out_ref):
    g8 = f8_ref[...]
    m = m_ref[...]
    f0 = jnp.where(m == 0, g8[:, 0:1],
                   jnp.where(m == 1, g8[:, 2:3],
                             jnp.where(m == 2, g8[:, 4:5], g8[:, 6:7])))
    f1 = jnp.where(m == 0, g8[:, 1:2],
                   jnp.where(m == 1, g8[:, 3:4],
                             jnp.where(m == 2, g8[:, 5:6], g8[:, 7:8])))
    w1 = w1_ref[...]
    h = jnp.tanh(f0 * w1[0:1, :] + f1 * w1[1:2, :] + b1_ref[...])
    fe = jnp.dot(h, w2_ref[...], preferred_element_type=jnp.float32)
    x = id_ref[...] + fe + b2_ref[...]
    mu = jnp.mean(x, axis=-1, keepdims=True)
    var = jnp.mean((x - mu) * (x - mu), axis=-1, keepdims=True)
    out_ref[...] = (x - mu) * lax.rsqrt(var + 1e-5) * g_ref[...] + be_ref[...]


def _tc_fused(id_emb, f8, mvec, W1, b1, W2, b2, gamma, beta):
    grid = (N // BT,)
    full = lambda s: pl.BlockSpec(s, lambda i: (0, 0))
    return pl.pallas_call(
        _tc_body,
        grid=grid,
        in_specs=[
            pl.BlockSpec((BT, H), lambda i: (i, 0)),
            pl.BlockSpec((BT, FW), lambda i: (i, 0)),
            pl.BlockSpec((BT, 1), lambda i: (i, 0)),
            full((2, H)), full((1, H)), full((H, H)), full((1, H)),
            full((1, H)), full((1, H)),
        ],
        out_specs=pl.BlockSpec((BT, H), lambda i: (i, 0)),
        out_shape=jax.ShapeDtypeStruct((N, H), jnp.float32),
    )(id_emb, f8, mvec, W1, b1, W2, b2, gamma, beta)


@jax.jit
def kernel(item_seq, id_table, freq_table, W1, b1, W2, b2, gamma, beta):
    idx = item_seq.reshape(NW, K, CHUNK)
    idx4 = idx >> 2
    freq8 = freq_table.reshape(NUM_ROWS8, FW)
    id_emb, f8 = _sc_gather_call()(idx, idx4, id_table, freq8)
    mvec = (item_seq.reshape(N, 1) & 3)
    out = _tc_fused(id_emb, f8, mvec, W1, b1.reshape(1, H), W2,
                    b2.reshape(1, H), gamma.reshape(1, H), beta.reshape(1, H))
    return out.reshape(B, L, H)


# speedup vs baseline: 3.0937x; 1.2121x over previous
import jax
import jax.numpy as jnp
from jax import lax
from jax.experimental import pallas as pl
from jax.experimental.pallas import tpu as pltpu
from jax.experimental.pallas import tpu_sc as plsc

B, L, H = 4096, 200, 64
N = B * L
NC, NS = 2, 16
NW = NC * NS
PER_W = N // NW
CHUNK = 128
K = PER_W // CHUNK
FW = 8
NUM_ROWS8 = 1000000 * 2 // FW


def _sc_gather_body(idx_hbm, id_tab, freq8_tab, id_out, fm_out,
                    idx_v, idxc_v, idx4_v, rows_v, frows_v, fmask_v,
                    sem_id, sem_fr):
    wid = lax.axis_index("s") * NC + lax.axis_index("c")
    pltpu.sync_copy(idx_hbm.at[wid], idx_v)

    iota = lax.iota(jnp.int32, 16)
    tw_off = iota >> 3
    colv = iota & 7
    keepcol = colv >> 1

    def body(j, carry):
        base = wid * PER_W + j * CHUNK
        for l in range(8):
            v = idx_v[j, pl.ds(l * 16, 16)]
            idxc_v[pl.ds(l * 16, 16)] = v
            idx4_v[pl.ds(l * 16, 16)] = v >> 2
        cp1 = pltpu.async_copy(id_tab.at[idx_v.at[j]], rows_v, sem_id)
        cp2 = pltpu.async_copy(freq8_tab.at[idx4_v], frows_v, sem_fr)
        cp1.wait()
        pltpu.sync_copy(rows_v, id_out.at[pl.ds(base, CHUNK)])
        cp2.wait()
        for i in range(64):
            twv = tw_off + 2 * i
            tok = plsc.load_gather(idxc_v, [twv])
            keep = keepcol == (tok & 3)
            row = plsc.load_gather(frows_v, [twv, colv])
            fmask_v[i] = jnp.where(keep, row, 0.0)
        rr0 = base // 2
        row0 = 128 * (rr0 // 1024) + rr0 % 128
        km = (rr0 % 1024) // 128
        pltpu.sync_copy(fmask_v, fm_out.at[pl.ds(row0, 64), pl.ds(16 * km, 16)])
        return carry

    lax.fori_loop(0, K, body, 0)


_SC_CACHE = {}


def _sc_gather_call():
    if "k" not in _SC_CACHE:
        _SC_CACHE["k"] = pl.kernel(
            _sc_gather_body,
            out_type=(jax.ShapeDtypeStruct((N, H), jnp.float32),
                      jax.ShapeDtypeStruct((N // 16, 128), jnp.float32)),
            mesh=plsc.VectorSubcoreMesh(core_axis_name="c", subcore_axis_name="s"),
            compiler_params=pltpu.CompilerParams(use_tc_tiling_on_sc=False,
                                                 needs_layout_passes=False),
            scratch_types=[
                pltpu.VMEM((K, CHUNK), jnp.int32),
                pltpu.VMEM((CHUNK,), jnp.int32),
                pltpu.VMEM((CHUNK,), jnp.int32),
                pltpu.VMEM((CHUNK, H), jnp.float32),
                pltpu.VMEM((CHUNK, FW), jnp.float32),
                pltpu.VMEM((64, 16), jnp.float32),
                pltpu.SemaphoreType.DMA,
                pltpu.SemaphoreType.DMA,
            ],
        )
    return _SC_CACHE["k"]


BT = 2048
BTH = BT // 2


def _tc_body(id_ref, fm_ref, w1_ref, b1_ref, w2_ref, b2_ref, g_ref,
             be_ref, out_ref):
    fm = fm_ref[...]
    w1 = w1_ref[...]
    b1 = b1_ref[...]
    w2 = w2_ref[...]
    hes, hos = [], []
    for k in range(8):
        c = 16 * k
        f0e = fm[:, c + 0:c + 1] + fm[:, c + 2:c + 3] + fm[:, c + 4:c + 5] + fm[:, c + 6:c + 7]
        f1e = fm[:, c + 1:c + 2] + fm[:, c + 3:c + 4] + fm[:, c + 5:c + 6] + fm[:, c + 7:c + 8]
        f0o = fm[:, c + 8:c + 9] + fm[:, c + 10:c + 11] + fm[:, c + 12:c + 13] + fm[:, c + 14:c + 15]
        f1o = fm[:, c + 9:c + 10] + fm[:, c + 11:c + 12] + fm[:, c + 13:c + 14] + fm[:, c + 15:c + 16]
        hes.append(jnp.tanh(f0e * w1[0:1, :] + f1e * w1[1:2, :] + b1))
        hos.append(jnp.tanh(f0o * w1[0:1, :] + f1o * w1[1:2, :] + b1))
    he = jnp.concatenate(hes, axis=0)
    ho = jnp.concatenate(hos, axis=0)
    fee = jnp.dot(he, w2, preferred_element_type=jnp.float32)
    feo = jnp.dot(ho, w2, preferred_element_type=jnp.float32)
    idp = id_ref[...]
    b2 = b2_ref[...]
    xe = idp[:, 0:H] + fee + b2
    xo = idp[:, H:128] + feo + b2

    def _ln(x):
        mu = jnp.mean(x, axis=-1, keepdims=True)
        var = jnp.mean((x - mu) * (x - mu), axis=-1, keepdims=True)
        return (x - mu) * lax.rsqrt(var + 1e-5) * g_ref[...] + be_ref[...]

    out_ref[...] = jnp.concatenate([_ln(xe), _ln(xo)], axis=1)


def _tc_fused(id2, fm, W1, b1, W2, b2, gamma, beta):
    grid = (N // BT,)
    full = lambda s: pl.BlockSpec(s, lambda i: (0, 0))
    return pl.pallas_call(
        _tc_body,
        grid=grid,
        in_specs=[
            pl.BlockSpec((BTH, 128), lambda i: (i, 0)),
            pl.BlockSpec((BT // 16, 128), lambda i: (i, 0)),
            full((2, H)), full((1, H)), full((H, H)), full((1, H)),
            full((1, H)), full((1, H)),
        ],
        out_specs=pl.BlockSpec((BTH, 128), lambda i: (i, 0)),
        out_shape=jax.ShapeDtypeStruct((N // 2, 128), jnp.float32),
    )(id2, fm, W1, b1, W2, b2, gamma, beta)


@jax.jit
def kernel(item_seq, id_table, freq_table, W1, b1, W2, b2, gamma, beta):
    idx = item_seq.reshape(NW, K, CHUNK)
    freq8 = freq_table.reshape(NUM_ROWS8, FW)
    id_emb, fm = _sc_gather_call()(idx, id_table, freq8)
    id2 = id_emb.reshape(N // 2, 128)
    out = _tc_fused(id2, fm, W1, b1.reshape(1, H), W2, b2.reshape(1, H),
                    gamma.reshape(1, H), beta.reshape(1, H))
    return out.reshape(B, L, H)


# speedup vs baseline: 4.5179x; 1.4604x over previous
import jax
import jax.numpy as jnp
from jax import lax
from jax.experimental import pallas as pl
from jax.experimental.pallas import tpu as pltpu
from jax.experimental.pallas import tpu_sc as plsc
B, L, H = 4096, 200, 64
N = B * L
NC, NS = 2, 16
NW = NC * NS
PER_W = N // NW
CHUNK = 128
K = PER_W // CHUNK
FW = 8
NROWS8 = 1000000 // FW


def _sc_gather_body(idx_hbm, id_tab, f0_tab, f1_tab, id_out, fm_out,
                    idx_v, idxc_v, idx8_v, rows_v, d0_v, d1_v, fmask_v,
                    sem_id, sem_f0, sem_f1):
    wid = lax.axis_index("s") * NC + lax.axis_index("c")
    pltpu.sync_copy(idx_hbm.at[wid], idx_v)

    iota = lax.iota(jnp.int32, 16)
    tv_off = (iota >> 1) & 1
    odd = (iota & 1) == 0
    low4 = iota < 4

    def body(j, carry):
        base = wid * PER_W + j * CHUNK
        for l in range(8):
            v = idx_v[j, pl.ds(l * 16, 16)]
            idxc_v[pl.ds(l * 16, 16)] = v
            idx8_v[pl.ds(l * 16, 16)] = v >> 3
        cp1 = pltpu.async_copy(id_tab.at[idx_v.at[j]], rows_v, sem_id)
        cp2 = pltpu.async_copy(f0_tab.at[idx8_v], d0_v, sem_f0)
        cp3 = pltpu.async_copy(f1_tab.at[idx8_v], d1_v, sem_f1)
        cp1.wait()
        pltpu.sync_copy(rows_v, id_out.at[pl.ds(base, CHUNK)])
        cp2.wait()
        cp3.wait()
        for i in range(64):
            tv = tv_off + 2 * i
            m = plsc.load_gather(idxc_v, [tv]) & 7
            a = plsc.load_gather(d0_v, [tv, m])
            bb = plsc.load_gather(d1_v, [tv, m])
            v = jnp.where(odd, a, bb)
            fmask_v[i] = jnp.where(low4, v, 0.0)
        rr0 = base // 2
        row0 = 128 * (rr0 // 1024) + rr0 % 128
        km = (rr0 % 1024) // 128
        pltpu.sync_copy(fmask_v, fm_out.at[pl.ds(row0, 64), pl.ds(16 * km, 16)])
        return carry

    lax.fori_loop(0, K, body, 0)


_SC_CACHE = {}


def _sc_gather_call():
    if "k" not in _SC_CACHE:
        _SC_CACHE["k"] = pl.kernel(
            _sc_gather_body,
            out_type=(jax.ShapeDtypeStruct((N, H), jnp.float32),
                      jax.ShapeDtypeStruct((N // 16, 128), jnp.float32)),
            mesh=plsc.VectorSubcoreMesh(core_axis_name="c", subcore_axis_name="s"),
            compiler_params=pltpu.CompilerParams(use_tc_tiling_on_sc=False,
                                                 needs_layout_passes=False),
            scratch_types=[
                pltpu.VMEM((K, CHUNK), jnp.int32),
                pltpu.VMEM((CHUNK,), jnp.int32),
                pltpu.VMEM((CHUNK,), jnp.int32),
                pltpu.VMEM((CHUNK, H), jnp.float32),
                pltpu.VMEM((CHUNK, FW), jnp.float32),
                pltpu.VMEM((CHUNK, FW), jnp.float32),
                pltpu.VMEM((64, 16), jnp.float32),
                pltpu.SemaphoreType.DMA,
                pltpu.SemaphoreType.DMA,
                pltpu.SemaphoreType.DMA,
            ],
        )
    return _SC_CACHE["k"]


BT = 2048
BTH = BT // 2


def _tc_body(id_ref, fm_ref, w1_ref, b1_ref, w2_ref, b2_ref, g_ref,
             be_ref, out_ref):
    fm = fm_ref[...]
    w1 = w1_ref[...]
    b1 = b1_ref[...]
    w2 = w2_ref[...]
    hes, hos = [], []
    for k in range(8):
        c = 16 * k
        f0e = fm[:, c + 0:c + 1]
        f1e = fm[:, c + 1:c + 2]
        f0o = fm[:, c + 2:c + 3]
        f1o = fm[:, c + 3:c + 4]
        hes.append(jnp.tanh(f0e * w1[0:1, :] + f1e * w1[1:2, :] + b1))
        hos.append(jnp.tanh(f0o * w1[0:1, :] + f1o * w1[1:2, :] + b1))
    he = jnp.concatenate(hes, axis=0)
    ho = jnp.concatenate(hos, axis=0)
    fee = jnp.dot(he, w2, preferred_element_type=jnp.float32)
    feo = jnp.dot(ho, w2, preferred_element_type=jnp.float32)
    idp = id_ref[...]
    b2 = b2_ref[...]
    xe = idp[:, 0:H] + fee + b2
    xo = idp[:, H:128] + feo + b2

    def _ln(x):
        mu = jnp.mean(x, axis=-1, keepdims=True)
        var = jnp.mean((x - mu) * (x - mu), axis=-1, keepdims=True)
        return (x - mu) * lax.rsqrt(var + 1e-5) * g_ref[...] + be_ref[...]

    out_ref[...] = jnp.concatenate([_ln(xe), _ln(xo)], axis=1)


def _tc_fused(id2, fm, W1, b1, W2, b2, gamma, beta):
    grid = (N // BT,)
    full = lambda s: pl.BlockSpec(s, lambda i: (0, 0))
    return pl.pallas_call(
        _tc_body,
        grid=grid,
        in_specs=[
            pl.BlockSpec((BTH, 128), lambda i: (i, 0)),
            pl.BlockSpec((BT // 16, 128), lambda i: (i, 0)),
            full((2, H)), full((1, H)), full((H, H)), full((1, H)),
            full((1, H)), full((1, H)),
        ],
        out_specs=pl.BlockSpec((BTH, 128), lambda i: (i, 0)),
        out_shape=jax.ShapeDtypeStruct((N // 2, 128), jnp.float32),
    )(id2, fm, W1, b1, W2, b2, gamma, beta)


@jax.jit
def kernel(item_seq, id_table, freq_table, W1, b1, W2, b2, gamma, beta):
    idx = item_seq.reshape(NW, K, CHUNK)
    f0_tab = freq_table[:, 0].reshape(NROWS8, FW)
    f1_tab = freq_table[:, 1].reshape(NROWS8, FW)
    id_emb, fm = _sc_gather_call()(idx, id_table, f0_tab, f1_tab)
    id2 = id_emb.reshape(N // 2, 128)
    out = _tc_fused(id2, fm, W1, b1.reshape(1, H), W2, b2.reshape(1, H),
                    gamma.reshape(1, H), beta.reshape(1, H))
    return out.reshape(B, L, H)
